# baseline (speedup 1.0000x reference)
import functools

import jax
import jax.numpy as jnp
from jax import lax
from jax.experimental import pallas as pl
from jax.experimental.pallas import tpu as pltpu
from jax.experimental.pallas import tpu_sc as plsc

N = 10000
NP = 10240
E = 320000
EP = 327680
D = 128
H = 128
F = 64
C = 2
NC, NS = 2, 16
NW = NC * NS
CH = 128
NCHW = EP // CH // NW
NBUF = 2
PFD = 1
STRIPE = NP // NS
BLK = 512
G = NP // BLK

_mesh = plsc.VectorSubcoreMesh(
    core_axis_name="c", subcore_axis_name="s", num_cores=NC, num_subcores=NS)


def _lrelu(v):
    return jnp.where(v > 0, v, 0.01 * v)


@functools.partial(
    pl.kernel,
    out_type=jax.ShapeDtypeStruct((NC, NP), jnp.float32),
    mesh=_mesh,
    compiler_params=pltpu.CompilerParams(use_tc_tiling_on_sc=False),
    scratch_types=[
        pltpu.VMEM((NCHW, CH), jnp.int32),
        pltpu.VMEM((CH,), jnp.float32),
        pltpu.VMEM((STRIPE,), jnp.float32),
        pltpu.VMEM_SHARED((NP,), jnp.float32),
        pltpu.SemaphoreType.DMA,
    ],
)
def _deg_kernel(dst_hbm, out_hbm, didx, ones_v, zeros_v, acc, ssem):
    c = lax.axis_index("c")
    s = lax.axis_index("s")
    wid = s * NC + c

    def fill_ones(i, _):
        ones_v[pl.ds(i * 16, 16)] = jnp.ones((16,), jnp.float32)
        return 0
    lax.fori_loop(0, CH // 16, fill_ones, 0)

    def fill_zeros(i, _):
        zeros_v[pl.ds(i * 16, 16)] = jnp.zeros((16,), jnp.float32)
        return 0
    lax.fori_loop(0, STRIPE // 16, fill_zeros, 0)

    pltpu.sync_copy(zeros_v, acc.at[pl.ds(s * STRIPE, STRIPE)])
    pltpu.sync_copy(dst_hbm.at[pl.ds(wid * NCHW, NCHW)], didx)
    plsc.subcore_barrier()

    def swait():
        pltpu.make_async_copy(ones_v, acc.at[didx.at[0]], ssem).wait()

    DEPTH = 8

    def fire(k, _):
        pltpu.async_copy(ones_v, acc.at[didx.at[k]], ssem, add=True)

        @pl.when(k >= DEPTH)
        def _():
            swait()
        return 0
    lax.fori_loop(0, NCHW, fire, 0)

    def drain(k, _):
        swait()
        return 0
    lax.fori_loop(0, DEPTH, drain, 0)

    plsc.subcore_barrier()
    pltpu.sync_copy(acc.at[pl.ds(s * STRIPE, STRIPE)],
                    out_hbm.at[c, pl.ds(s * STRIPE, STRIPE)])


@functools.partial(
    pl.kernel,
    out_type=jax.ShapeDtypeStruct((NC, NP, F), jnp.float32),
    mesh=_mesh,
    compiler_params=pltpu.CompilerParams(use_tc_tiling_on_sc=False),
    scratch_types=[
        pltpu.VMEM((NCHW, CH), jnp.int32),
        pltpu.VMEM((NCHW, CH), jnp.int32),
        pltpu.VMEM((NBUF, CH, F), jnp.float32),
        pltpu.VMEM_SHARED((NP, F), jnp.float32),
        pltpu.VMEM_SHARED((NP, F), jnp.float32),
        pltpu.SemaphoreType.DMA,
        pltpu.SemaphoreType.DMA,
    ],
)
def _scatter_kernel(y_hbm, src_hbm, dst_hbm, out_hbm,
                    sidx, didx, rows, ytab, acc, gsem, ssem):
    c = lax.axis_index("c")
    s = lax.axis_index("s")
    wid = s * NC + c

    pltpu.sync_copy(y_hbm.at[pl.ds(s * STRIPE, STRIPE)],
                    ytab.at[pl.ds(s * STRIPE, STRIPE)])

    def fz(i, _):
        r = i // (F // 16)
        j = lax.rem(i, F // 16)
        rows[0, r, pl.ds(j * 16, 16)] = jnp.zeros((16,), jnp.float32)
        return 0
    lax.fori_loop(0, CH * (F // 16), fz, 0)

    def zcopy(i, _):
        pltpu.sync_copy(rows.at[0], acc.at[pl.ds(s * STRIPE + i * CH, CH)])
        return 0
    lax.fori_loop(0, STRIPE // CH, zcopy, 0)

    pltpu.sync_copy(src_hbm.at[pl.ds(wid * NCHW, NCHW)], sidx)
    pltpu.sync_copy(dst_hbm.at[pl.ds(wid * NCHW, NCHW)], didx)
    plsc.subcore_barrier()

    def gwait(b):
        pltpu.make_async_copy(ytab.at[sidx.at[0]], rows.at[b], gsem).wait()

    def swait():
        pltpu.make_async_copy(rows.at[0], acc.at[didx.at[0]], ssem).wait()

    def gissue(k, b):
        @pl.when(lax.rem(k, 3) == 0)
        def _():
            pltpu.async_copy(y_hbm.at[sidx.at[k]], rows.at[b], gsem)

        @pl.when(lax.rem(k, 3) != 0)
        def _():
            pltpu.async_copy(ytab.at[sidx.at[k]], rows.at[b], gsem)

    for b in range(PFD):
        gissue(b, b)

    def outer(ko, _):
        for b in range(NBUF):
            k = ko * NBUF + b
            gwait(b)
            pltpu.async_copy(rows.at[b], acc.at[didx.at[k]], ssem, add=True)
            if b >= NBUF - PFD:
                swait()
            else:
                @pl.when(ko >= 1)
                def _():
                    swait()

            @pl.when(k + PFD < NCHW)
            def _():
                gissue(k + PFD, (b + PFD) % NBUF)
        return 0
    lax.fori_loop(0, NCHW // NBUF, outer, 0,)

    for _ in range(NBUF - PFD):
        swait()
    plsc.subcore_barrier()
    pltpu.sync_copy(acc.at[pl.ds(s * STRIPE, STRIPE)],
                    out_hbm.at[c, pl.ds(s * STRIPE, STRIPE)])


def _tc1_body(x_ref, w1_ref, b1_ref, w2_ref, b2_ref, wg1_ref, degp_ref,
              y1_ref):
    h = _lrelu(jnp.dot(x_ref[...], w1_ref[...],
                       preferred_element_type=jnp.float32) + b1_ref[...])
    h = _lrelu(jnp.dot(h, w2_ref[...],
                       preferred_element_type=jnp.float32) + b2_ref[...])
    d = degp_ref[...]
    dinv = lax.rsqrt(d[0] + d[1] + 1.0)
    y1_ref[...] = jnp.dot(h, wg1_ref[...],
                          preferred_element_type=jnp.float32) * dinv


def _tc2_body(sp_ref, y1_ref, degp_ref, bg1_ref, wg2_ref, y2_ref):
    d = degp_ref[...]
    dinv = lax.rsqrt(d[0] + d[1] + 1.0)
    sp = sp_ref[...]
    h3 = _lrelu(dinv * (sp[0] + sp[1] + y1_ref[...]) + bg1_ref[...])
    y2_ref[...] = jnp.dot(h3, wg2_ref[...],
                          preferred_element_type=jnp.float32) * dinv


def _tc3_body(sp_ref, y2_ref, degp_ref, bg2_ref, wc_ref, bc_ref,
              logits_ref, h_ref):
    d = degp_ref[...]
    dinv = lax.rsqrt(d[0] + d[1] + 1.0)
    sp = sp_ref[...]
    h4 = _lrelu(dinv * (sp[0] + sp[1] + y2_ref[...]) + bg2_ref[...])
    h_ref[...] = h4
    logits_ref[...] = jnp.dot(h4, wc_ref[...],
                              preferred_element_type=jnp.float32) + bc_ref[...]


_full = lambda shape: pl.BlockSpec(shape, lambda i: tuple(0 for _ in shape))
_rows = lambda w: pl.BlockSpec((BLK, w), lambda i: (i, 0))
_degs = pl.BlockSpec((2, BLK, 1), lambda i: (0, i, 0))
_parts = pl.BlockSpec((2, BLK, F), lambda i: (0, i, 0))

_tc1 = pl.pallas_call(
    _tc1_body,
    grid=(G,),
    in_specs=[_rows(D), _full((D, H)), _full((1, H)), _full((H, F)),
              _full((1, F)), _full((F, F)), _degs],
    out_specs=_rows(F),
    out_shape=jax.ShapeDtypeStruct((NP, F), jnp.float32),
)

_tc2 = pl.pallas_call(
    _tc2_body,
    grid=(G,),
    in_specs=[_parts, _rows(F), _degs, _full((1, F)), _full((F, F))],
    out_specs=_rows(F),
    out_shape=jax.ShapeDtypeStruct((NP, F), jnp.float32),
)

_tc3 = pl.pallas_call(
    _tc3_body,
    grid=(G,),
    in_specs=[_parts, _rows(F), _degs, _full((1, F)), _full((F, C)),
              _full((1, C))],
    out_specs=[_rows(C), _rows(F)],
    out_shape=[jax.ShapeDtypeStruct((NP, C), jnp.float32),
               jax.ShapeDtypeStruct((NP, F), jnp.float32)],
)


def kernel(x, edge_index, W1, b1, W2, b2, Wg1, bg1, Wg2, bg2, Wc, bc):
    src = edge_index[0].astype(jnp.int32)
    dst = edge_index[1].astype(jnp.int32)
    src2d = jnp.pad(src, (0, EP - E)).reshape(EP // CH, CH)
    dst2d = jnp.pad(dst, (0, EP - E),
                    constant_values=N).reshape(EP // CH, CH)
    xp = jnp.pad(x, ((0, NP - N), (0, 0)))

    degp = _deg_kernel(dst2d)
    degp3 = degp.reshape(NC, NP, 1)

    y1 = _tc1(xp, W1, b1.reshape(1, H), W2, b2.reshape(1, F), Wg1, degp3)
    s1p = _scatter_kernel(y1, src2d, dst2d)
    y2 = _tc2(s1p, y1, degp3, bg1.reshape(1, F), Wg2)
    s2p = _scatter_kernel(y2, src2d, dst2d)
    logits, h = _tc3(s2p, y2, degp3, bg2.reshape(1, F), Wc,
                     bc.reshape(1, C))
    return logits[:N], h[:N]


# speedup vs baseline: 1.3283x; 1.3283x over previous
import functools

import jax
import jax.numpy as jnp
from jax import lax
from jax.experimental import pallas as pl
from jax.experimental.pallas import tpu as pltpu
from jax.experimental.pallas import tpu_sc as plsc

N = 10000
NP = 10240
E = 320000
EP = 327680
D = 128
H = 128
F = 64
C = 2
NC, NS = 2, 16
NW = NC * NS
CH = 128
NCHW = EP // CH // NW
NBUF = 2
PFD = 1
STRIPE = NP // NS
BLK = 512
G = NP // BLK

_mesh = plsc.VectorSubcoreMesh(
    core_axis_name="c", subcore_axis_name="s", num_cores=NC, num_subcores=NS)


def _lrelu(v):
    return jnp.where(v > 0, v, 0.01 * v)


@functools.partial(
    pl.kernel,
    out_type=jax.ShapeDtypeStruct((NC, NP), jnp.float32),
    mesh=_mesh,
    compiler_params=pltpu.CompilerParams(use_tc_tiling_on_sc=False),
    scratch_types=[
        pltpu.VMEM((NCHW, CH), jnp.int32),
        pltpu.VMEM((CH,), jnp.float32),
        pltpu.VMEM((STRIPE,), jnp.float32),
        pltpu.VMEM_SHARED((NP,), jnp.float32),
        pltpu.SemaphoreType.DMA,
    ],
)
def _deg_kernel(dst_hbm, out_hbm, didx, ones_v, zeros_v, acc, ssem):
    c = lax.axis_index("c")
    s = lax.axis_index("s")
    wid = s * NC + c

    def fill_ones(i, _):
        ones_v[pl.ds(i * 16, 16)] = jnp.ones((16,), jnp.float32)
        return 0
    lax.fori_loop(0, CH // 16, fill_ones, 0)

    def fill_zeros(i, _):
        zeros_v[pl.ds(i * 16, 16)] = jnp.zeros((16,), jnp.float32)
        return 0
    lax.fori_loop(0, STRIPE // 16, fill_zeros, 0)

    pltpu.sync_copy(zeros_v, acc.at[pl.ds(s * STRIPE, STRIPE)])
    pltpu.sync_copy(dst_hbm.at[pl.ds(wid * NCHW, NCHW)], didx)
    plsc.subcore_barrier()

    def swait():
        pltpu.make_async_copy(ones_v, acc.at[didx.at[0]], ssem).wait()

    DEPTH = 8

    def fire(k, _):
        pltpu.async_copy(ones_v, acc.at[didx.at[k]], ssem, add=True)

        @pl.when(k >= DEPTH)
        def _():
            swait()
        return 0
    lax.fori_loop(0, NCHW, fire, 0)

    def drain(k, _):
        swait()
        return 0
    lax.fori_loop(0, DEPTH, drain, 0)

    plsc.subcore_barrier()
    pltpu.sync_copy(acc.at[pl.ds(s * STRIPE, STRIPE)],
                    out_hbm.at[c, pl.ds(s * STRIPE, STRIPE)])


@functools.partial(
    pl.kernel,
    out_type=jax.ShapeDtypeStruct((NC, NP, F), jnp.float32),
    mesh=_mesh,
    compiler_params=pltpu.CompilerParams(use_tc_tiling_on_sc=False),
    scratch_types=[
        pltpu.VMEM((NCHW, CH), jnp.int32),
        pltpu.VMEM((NCHW, CH), jnp.int32),
        pltpu.VMEM((NBUF, CH, F), jnp.float32),
        pltpu.VMEM_SHARED((NP, F), jnp.float32),
        pltpu.VMEM_SHARED((NP, F), jnp.float32),
        pltpu.SemaphoreType.DMA,
        pltpu.SemaphoreType.DMA,
    ],
)
def _scatter_kernel(y_hbm, src_hbm, dst_hbm, out_hbm,
                    sidx, didx, rows, ytab, acc, gsem, ssem):
    c = lax.axis_index("c")
    s = lax.axis_index("s")
    wid = s * NC + c

    pltpu.sync_copy(y_hbm.at[pl.ds(s * STRIPE, STRIPE)],
                    ytab.at[pl.ds(s * STRIPE, STRIPE)])

    def fz(i, _):
        r = i // (F // 16)
        j = lax.rem(i, F // 16)
        rows[0, r, pl.ds(j * 16, 16)] = jnp.zeros((16,), jnp.float32)
        return 0
    lax.fori_loop(0, CH * (F // 16), fz, 0)

    def zcopy(i, _):
        pltpu.sync_copy(rows.at[0], acc.at[pl.ds(s * STRIPE + i * CH, CH)])
        return 0
    lax.fori_loop(0, STRIPE // CH, zcopy, 0)

    pltpu.sync_copy(src_hbm.at[pl.ds(wid * NCHW, NCHW)], sidx)
    pltpu.sync_copy(dst_hbm.at[pl.ds(wid * NCHW, NCHW)], didx)
    plsc.subcore_barrier()

    def gwait(b):
        pltpu.make_async_copy(ytab.at[sidx.at[0]], rows.at[b], gsem).wait()

    def swait():
        pltpu.make_async_copy(rows.at[0], acc.at[didx.at[0]], ssem).wait()

    def gissue(k, b):
        pltpu.async_copy(ytab.at[sidx.at[k]], rows.at[b], gsem)

    for b in range(PFD):
        gissue(b, b)

    def outer(ko, _):
        for b in range(NBUF):
            k = ko * NBUF + b
            gwait(b)
            pltpu.async_copy(rows.at[b], acc.at[didx.at[k]], ssem, add=True)
            if b >= NBUF - PFD:
                swait()
            else:
                @pl.when(ko >= 1)
                def _():
                    swait()

            @pl.when(k + PFD < NCHW)
            def _():
                gissue(k + PFD, (b + PFD) % NBUF)
        return 0
    lax.fori_loop(0, NCHW // NBUF, outer, 0,)

    for _ in range(NBUF - PFD):
        swait()
    plsc.subcore_barrier()
    pltpu.sync_copy(acc.at[pl.ds(s * STRIPE, STRIPE)],
                    out_hbm.at[c, pl.ds(s * STRIPE, STRIPE)])


def _tc1_body(x_ref, w1_ref, b1_ref, w2_ref, b2_ref, wg1_ref, degp_ref,
              y1_ref):
    h = _lrelu(jnp.dot(x_ref[...], w1_ref[...],
                       preferred_element_type=jnp.float32) + b1_ref[...])
    h = _lrelu(jnp.dot(h, w2_ref[...],
                       preferred_element_type=jnp.float32) + b2_ref[...])
    d = degp_ref[...]
    dinv = lax.rsqrt(d[0] + d[1] + 1.0)
    y1_ref[...] = jnp.dot(h, wg1_ref[...],
                          preferred_element_type=jnp.float32) * dinv


def _tc2_body(sp_ref, y1_ref, degp_ref, bg1_ref, wg2_ref, y2_ref):
    d = degp_ref[...]
    dinv = lax.rsqrt(d[0] + d[1] + 1.0)
    sp = sp_ref[...]
    h3 = _lrelu(dinv * (sp[0] + sp[1] + y1_ref[...]) + bg1_ref[...])
    y2_ref[...] = jnp.dot(h3, wg2_ref[...],
                          preferred_element_type=jnp.float32) * dinv


def _tc3_body(sp_ref, y2_ref, degp_ref, bg2_ref, wc_ref, bc_ref,
              logits_ref, h_ref):
    d = degp_ref[...]
    dinv = lax.rsqrt(d[0] + d[1] + 1.0)
    sp = sp_ref[...]
    h4 = _lrelu(dinv * (sp[0] + sp[1] + y2_ref[...]) + bg2_ref[...])
    h_ref[...] = h4
    logits_ref[...] = jnp.dot(h4, wc_ref[...],
                              preferred_element_type=jnp.float32) + bc_ref[...]


_full = lambda shape: pl.BlockSpec(shape, lambda i: tuple(0 for _ in shape))
_rows = lambda w: pl.BlockSpec((BLK, w), lambda i: (i, 0))
_degs = pl.BlockSpec((2, BLK, 1), lambda i: (0, i, 0))
_parts = pl.BlockSpec((2, BLK, F), lambda i: (0, i, 0))

_tc1 = pl.pallas_call(
    _tc1_body,
    grid=(G,),
    in_specs=[_rows(D), _full((D, H)), _full((1, H)), _full((H, F)),
              _full((1, F)), _full((F, F)), _degs],
    out_specs=_rows(F),
    out_shape=jax.ShapeDtypeStruct((NP, F), jnp.float32),
)

_tc2 = pl.pallas_call(
    _tc2_body,
    grid=(G,),
    in_specs=[_parts, _rows(F), _degs, _full((1, F)), _full((F, F))],
    out_specs=_rows(F),
    out_shape=jax.ShapeDtypeStruct((NP, F), jnp.float32),
)

_tc3 = pl.pallas_call(
    _tc3_body,
    grid=(G,),
    in_specs=[_parts, _rows(F), _degs, _full((1, F)), _full((F, C)),
              _full((1, C))],
    out_specs=[_rows(C), _rows(F)],
    out_shape=[jax.ShapeDtypeStruct((NP, C), jnp.float32),
               jax.ShapeDtypeStruct((NP, F), jnp.float32)],
)


def kernel(x, edge_index, W1, b1, W2, b2, Wg1, bg1, Wg2, bg2, Wc, bc):
    src = edge_index[0].astype(jnp.int32)
    dst = edge_index[1].astype(jnp.int32)
    src2d = jnp.pad(src, (0, EP - E)).reshape(EP // CH, CH)
    dst2d = jnp.pad(dst, (0, EP - E),
                    constant_values=N).reshape(EP // CH, CH)
    xp = jnp.pad(x, ((0, NP - N), (0, 0)))

    degp = _deg_kernel(dst2d)
    degp3 = degp.reshape(NC, NP, 1)

    y1 = _tc1(xp, W1, b1.reshape(1, H), W2, b2.reshape(1, F), Wg1, degp3)
    s1p = _scatter_kernel(y1, src2d, dst2d)
    y2 = _tc2(s1p, y1, degp3, bg1.reshape(1, F), Wg2)
    s2p = _scatter_kernel(y2, src2d, dst2d)
    logits, h = _tc3(s2p, y2, degp3, bg2.reshape(1, F), Wc,
                     bc.reshape(1, C))
    return logits[:N], h[:N]
